# baseline (speedup 1.0000x reference)
import jax
import jax.numpy as jnp
from jax import lax
from jax.experimental import pallas as pl
from jax.experimental.pallas import tpu as pltpu
from jax.experimental.pallas import tpu_sc as plsc

_B, _H, _W, _P = 8, 512, 512, 100000
_NW = 32
_TCOL = 128
_FULLT = _P // _TCOL
_TPW = _FULLT // _NW
_NEXTRA = _FULLT - _NW * _TPW
_PARTW = _P - _FULLT * _TCOL
_CT = 4
_CCOL = _CT * _TCOL
_CE = _B * _CCOL
_NCHUNK = _TPW // _CT
_XE = _B * _TCOL
_PE = _B * _PARTW


def _loss16(za, zb, t):
    d = za - zb
    m = t * t
    u = -(d * t)
    e = jnp.exp(-jnp.abs(u))
    s = e / (2.0 + e)
    s2 = s * s
    p = s2 * (1.0 / 7.0) + (1.0 / 5.0)
    p = s2 * p + (1.0 / 3.0)
    p = s2 * p + 1.0
    ln1pe = 2.0 * s * p
    sp = jnp.maximum(u, 0.0) + ln1pe
    return m * sp + (1.0 - m) * (d * d)


def _sc_body(img, xa, ya, xb, yb, tg, xpa, ypa, xpb, ypb, tgp, out,
             xaA, yaA, xbA, ybA, xaB, yaB, xbB, ybB, tgA, tgB,
             iaA, ibA, iaB, ibB, zaA, zbA, zaB, zbB,
             xt, yt, x2t, y2t, tgt_v, iat, ibt, zat, zbt, acc_v,
             xp_v, yp_v, xp2_v, yp2_v, tgp_v,
             ssemA, ssemB, tsemA, tsemB, gsemA, gsemB, tailsem,
             hsemA, hsemB, tailsem2):
    c_ax = lax.axis_index("c")
    s_ax = lax.axis_index("s")
    wid = c_ax * 16 + s_ax
    base_col = wid * (_TPW * _TCOL)

    xy_srcs = (xa, ya, xb, yb)

    def fire_xy(coff, ncols, bufs, sem):
        for src, dst in zip(xy_srcs, bufs):
            pltpu.async_copy(
                src.at[pl.ds(0, _B), pl.ds(coff, ncols)],
                dst.at[pl.ds(0, _B), pl.ds(0, ncols)], sem)

    def wait_xy(ncols, bufs, sem):
        for src, dst in zip(xy_srcs, bufs):
            pltpu.make_async_copy(
                src.at[pl.ds(0, _B), pl.ds(0, ncols)],
                dst.at[pl.ds(0, _B), pl.ds(0, ncols)], sem).wait()

    def fire_tg(coff, ncols, buf, sem):
        pltpu.async_copy(
            tg.at[pl.ds(0, _B), pl.ds(coff, ncols)],
            buf.at[pl.ds(0, _B), pl.ds(0, ncols)], sem)

    def wait_tg(ncols, buf, sem):
        pltpu.make_async_copy(
            tg.at[pl.ds(0, _B), pl.ds(0, ncols)],
            buf.at[pl.ds(0, _B), pl.ds(0, ncols)], sem).wait()

    def compute_idx(ncols, bufs, ia_d, ib_d):
        xab, yab, xbb, ybb = bufs
        for r in range(_B):
            boff = r * (_H * _W)

            def ibody(i, _):
                sl = pl.ds(i * 16, 16)
                dl = pl.ds(r * ncols + i * 16, 16)
                ia_d[dl] = boff + lax.shift_left(xab[r, sl], 9) + yab[r, sl]
                ib_d[dl] = boff + lax.shift_left(xbb[r, sl], 9) + ybb[r, sl]
                return 0
            lax.fori_loop(0, ncols // 16, ibody, 0, unroll=4)

    def fire_gather(n, ia_d, ib_d, za_d, zb_d, sem, hsem):
        pltpu.async_copy(img.at[ia_d.at[pl.ds(0, n)]],
                         za_d.at[pl.ds(0, n)], sem)
        pltpu.async_copy(img.at[ib_d.at[pl.ds(0, n)]],
                         zb_d.at[pl.ds(0, n)], hsem)

    def wait_gather(n, ia_d, ib_d, za_d, zb_d, sem, hsem):
        pltpu.make_async_copy(img.at[ia_d.at[pl.ds(0, n)]],
                              za_d.at[pl.ds(0, n)], sem).wait()
        pltpu.make_async_copy(img.at[ib_d.at[pl.ds(0, n)]],
                              zb_d.at[pl.ds(0, n)], hsem).wait()

    def compute(ncols, tg_d, za_d, zb_d, acc):
        for r in range(_B):
            def cbody(i, a):
                sl = pl.ds(i * 16, 16)
                dl = pl.ds(r * ncols + i * 16, 16)
                return a + _loss16(za_d[dl], zb_d[dl], tg_d[r, sl])
            acc = lax.fori_loop(0, ncols // 16, cbody, acc, unroll=4)
        return acc

    bufsA = (xaA, yaA, xbA, ybA)
    bufsB = (xaB, yaB, xbB, ybB)
    tbufs = (xt, yt, x2t, y2t)

    def coff(c):
        return base_col + c * _CCOL

    xoff = (_NW * _TPW + jnp.clip(wid, 0, _NEXTRA - 1)) * _TCOL
    fire_xy(xoff, _TCOL, tbufs, tailsem)
    fire_tg(xoff, _TCOL, tgt_v, tailsem2)
    wait_xy(_TCOL, tbufs, tailsem)
    compute_idx(_TCOL, tbufs, iat, ibt)
    fire_gather(_XE, iat, ibt, zat, zbt, tailsem, tailsem2)
    wait_tg(_TCOL, tgt_v, tailsem2)
    wait_gather(_XE, iat, ibt, zat, zbt, tailsem, tailsem2)
    tacc = compute(_TCOL, tgt_v, zat, zbt, jnp.zeros((16,), jnp.float32))
    acc0 = (wid < _NEXTRA).astype(jnp.float32) * tacc

    pltpu.sync_copy(xpa, xp_v)
    pltpu.sync_copy(ypa, yp_v)
    pltpu.sync_copy(xpb, xp2_v)
    pltpu.sync_copy(ypb, yp2_v)
    pltpu.sync_copy(tgp, tgp_v)
    for r in range(_B):
        boff = r * (_H * _W)
        for i in range(_PARTW // 16):
            dl = pl.ds(r * _PARTW + i * 16, 16)
            iat[dl] = boff + lax.shift_left(xp_v[dl], 9) + yp_v[dl]
            ibt[dl] = boff + lax.shift_left(xp2_v[dl], 9) + yp2_v[dl]
    fire_gather(_PE, iat, ibt, zat, zbt, tailsem, tailsem2)
    wait_gather(_PE, iat, ibt, zat, zbt, tailsem, tailsem2)

    def pbody(i, a):
        dl = pl.ds(i * 16, 16)
        return a + _loss16(zat[dl], zbt[dl], tgp_v[dl])
    tacc = lax.fori_loop(0, _PE // 16, pbody, jnp.zeros((16,), jnp.float32))
    acc0 = acc0 + (wid == (_NW - 1)).astype(jnp.float32) * tacc

    fire_xy(coff(0), _CCOL, bufsA, ssemA)
    fire_xy(coff(1), _CCOL, bufsB, ssemB)
    wait_xy(_CCOL, bufsA, ssemA)
    compute_idx(_CCOL, bufsA, iaA, ibA)
    fire_gather(_CE, iaA, ibA, zaA, zbA, gsemA, hsemA)
    fire_tg(coff(0), _CCOL, tgA, tsemA)

    def jbody(j, acc):
        c0 = 2 * j

        @pl.when(j <= (_NCHUNK // 2 - 2))
        def _():
            fire_xy(coff(c0 + 2), _CCOL, bufsA, ssemA)
        wait_xy(_CCOL, bufsB, ssemB)
        compute_idx(_CCOL, bufsB, iaB, ibB)
        fire_gather(_CE, iaB, ibB, zaB, zbB, gsemB, hsemB)
        fire_tg(coff(c0 + 1), _CCOL, tgB, tsemB)
        wait_gather(_CE, iaA, ibA, zaA, zbA, gsemA, hsemA)
        wait_tg(_CCOL, tgA, tsemA)
        acc = compute(_CCOL, tgA, zaA, zbA, acc)

        @pl.when(j <= (_NCHUNK // 2 - 2))
        def _():
            fire_xy(coff(c0 + 3), _CCOL, bufsB, ssemB)
            wait_xy(_CCOL, bufsA, ssemA)
            compute_idx(_CCOL, bufsA, iaA, ibA)
            fire_gather(_CE, iaA, ibA, zaA, zbA, gsemA, hsemA)
            fire_tg(coff(c0 + 2), _CCOL, tgA, tsemA)
        wait_gather(_CE, iaB, ibB, zaB, zbB, gsemB, hsemB)
        wait_tg(_CCOL, tgB, tsemB)
        acc = compute(_CCOL, tgB, zaB, zbB, acc)
        return acc

    acc = lax.fori_loop(0, _NCHUNK // 2, jbody, acc0)

    acc_v[...] = acc
    pltpu.sync_copy(acc_v, out.at[wid])


_depth_loss_sc = pl.kernel(
    _sc_body,
    out_type=jax.ShapeDtypeStruct((_NW, 16), jnp.float32),
    mesh=plsc.VectorSubcoreMesh(
        core_axis_name="c", subcore_axis_name="s", num_cores=2,
        num_subcores=16),
    scratch_types=(
        [pltpu.VMEM((_B, _CCOL), jnp.int32)] * 8
        + [pltpu.VMEM((_B, _CCOL), jnp.float32)] * 2
        + [pltpu.VMEM((_CE,), jnp.int32)] * 4
        + [pltpu.VMEM((_CE,), jnp.float32)] * 4
        + [pltpu.VMEM((_B, _TCOL), jnp.int32)] * 4
        + [pltpu.VMEM((_B, _TCOL), jnp.float32)]
        + [pltpu.VMEM((_XE,), jnp.int32)] * 2
        + [pltpu.VMEM((_XE,), jnp.float32)] * 2
        + [pltpu.VMEM((16,), jnp.float32)]
        + [pltpu.VMEM((_PE,), jnp.int32)] * 4
        + [pltpu.VMEM((_PE,), jnp.float32)]
        + [pltpu.SemaphoreType.DMA] * 10
    ),
)


def kernel(output, x_A, y_A, x_B, y_B, ordinal_relation):
    img = output.reshape(_B * _H * _W)
    pcol = _FULLT * _TCOL
    xa = x_A.astype(jnp.int32)
    ya = y_A.astype(jnp.int32)
    xb = x_B.astype(jnp.int32)
    yb = y_B.astype(jnp.int32)
    tg = ordinal_relation.astype(jnp.float32)
    partials = _depth_loss_sc(
        img, xa, ya, xb, yb, tg,
        xa[:, pcol:].reshape(_PE), ya[:, pcol:].reshape(_PE),
        xb[:, pcol:].reshape(_PE), yb[:, pcol:].reshape(_PE),
        tg[:, pcol:].reshape(_PE))
    return jnp.sum(partials) / _B


# speedup vs baseline: 1.0550x; 1.0550x over previous
import jax
import jax.numpy as jnp
from jax import lax
from jax.experimental import pallas as pl
from jax.experimental.pallas import tpu as pltpu
from jax.experimental.pallas import tpu_sc as plsc

_B, _H, _W, _P = 8, 512, 512, 100000
_Q = _B * _P
_NW = 32
_C = 4160
_NCHUNK = 6
_MAIN = _NW * _C * _NCHUNK
_TG = 128
_NTAIL = (_Q - _MAIN) // _TG
_IMGS_PER_CORE = (_B // 2) * _H * _W
_STRIPE = _IMGS_PER_CORE // 16
_BOUNCE = 8192
_NROUND = _STRIPE // _BOUNCE


def _loss16(za, zb, t):
    d = za - zb
    m = t * t
    u = -(d * t)
    e = jnp.exp(-jnp.abs(u))
    s = e / (2.0 + e)
    s2 = s * s
    p = s2 * (1.0 / 7.0) + (1.0 / 5.0)
    p = s2 * p + (1.0 / 3.0)
    p = s2 * p + 1.0
    ln1pe = 2.0 * s * p
    sp = jnp.maximum(u, 0.0) + ln1pe
    return m * sp + (1.0 - m) * (d * d)


def _sc_body(img, xa, ya, xb, yb, tg, out,
             xaA, yaA, xbA, ybA, xaB, yaB, xbB, ybB, tgA, tgB,
             iaA, ibA, iaB, ibB, zaA, zbA, zaB, zbB,
             xt, yt, x2t, y2t, tgt_v, iat, ibt, zat, zbt, acc_v,
             ssemA, ssemB, tsemA, tsemB, gsemA, gsemB, tailsem,
             hsemA, hsemB, tailsem2):
    c_ax = lax.axis_index("c")
    s_ax = lax.axis_index("s")
    wid = c_ax * 16 + s_ax
    base = wid * (_C * _NCHUNK)
    sbase = c_ax * _IMGS_PER_CORE

    xy_srcs = (xa, ya, xb, yb)

    def fire_xy(eb, bufs, sem):
        for src, dst in zip(xy_srcs, bufs):
            pltpu.async_copy(src.at[pl.ds(eb, _C)], dst, sem)

    def wait_xy(bufs, sem):
        for src, dst in zip(xy_srcs, bufs):
            pltpu.make_async_copy(src.at[pl.ds(0, _C)], dst, sem).wait()

    def fire_tg(eb, buf, sem):
        pltpu.async_copy(tg.at[pl.ds(eb, _C)], buf, sem)

    def wait_tg(buf, sem):
        pltpu.make_async_copy(tg.at[pl.ds(0, _C)], buf, sem).wait()

    def compute_idx(eb, n, bufs, ia_d, ib_d):
        xab, yab, xbb, ybb = bufs

        def ibody(i, _):
            sl = pl.ds(i * 16, 16)
            q = lax.iota(jnp.int32, 16) + (eb + i * 16)
            boff = lax.shift_left(lax.div(q, _P), 18)
            ia_d[sl] = boff + lax.shift_left(xab[sl], 9) + yab[sl]
            ib_d[sl] = boff + lax.shift_left(xbb[sl], 9) + ybb[sl]
            return 0
        lax.fori_loop(0, n // 16, ibody, 0, unroll=4)

    def fire_gather(ia_d, ib_d, za_d, zb_d, sem, hsem):
        pltpu.async_copy(img.at[ia_d], za_d, sem)
        pltpu.async_copy(img.at[ib_d], zb_d, hsem)

    def wait_gather(ia_d, ib_d, za_d, zb_d, sem, hsem):
        pltpu.make_async_copy(img.at[ia_d], za_d, sem).wait()
        pltpu.make_async_copy(img.at[ib_d], zb_d, hsem).wait()

    def compute(n, tg_d, za_d, zb_d, acc):
        def cbody(i, a):
            sl = pl.ds(i * 16, 16)
            return a + _loss16(za_d[sl], zb_d[sl], tg_d[sl])
        return lax.fori_loop(0, n // 16, cbody, acc, unroll=4)

    bufsA = (xaA, yaA, xbA, ybA)
    bufsB = (xaB, yaB, xbB, ybB)

    def ebs(c):
        return base + c * _C

    tw = jnp.clip(wid - (_NW - _NTAIL), 0, _NTAIL - 1)
    eb2 = _MAIN + tw * _TG
    pltpu.sync_copy(xa.at[pl.ds(eb2, _TG)], xt)
    pltpu.sync_copy(ya.at[pl.ds(eb2, _TG)], yt)
    pltpu.sync_copy(xb.at[pl.ds(eb2, _TG)], x2t)
    pltpu.sync_copy(yb.at[pl.ds(eb2, _TG)], y2t)
    pltpu.sync_copy(tg.at[pl.ds(eb2, _TG)], tgt_v)
    compute_idx(eb2, _TG, (xt, yt, x2t, y2t), iat, ibt)

    fire_xy(ebs(0), bufsA, ssemA)
    fire_xy(ebs(1), bufsB, ssemB)
    wait_xy(bufsA, ssemA)
    compute_idx(ebs(0), _C, bufsA, iaA, ibA)

    fire_gather(iat, ibt, zat, zbt, tailsem, tailsem2)
    fire_gather(iaA, ibA, zaA, zbA, gsemA, hsemA)
    fire_tg(ebs(0), tgA, tsemA)
    wait_gather(iat, ibt, zat, zbt, tailsem, tailsem2)
    tacc = compute(_TG, tgt_v, zat, zbt, jnp.zeros((16,), jnp.float32))
    wmask = (wid >= (_NW - _NTAIL)).astype(jnp.float32)
    acc0 = wmask * tacc

    def jbody(j, acc):
        c0 = 2 * j

        @pl.when(j <= (_NCHUNK // 2 - 2))
        def _():
            fire_xy(ebs(c0 + 2), bufsA, ssemA)
        wait_xy(bufsB, ssemB)
        compute_idx(ebs(c0 + 1), _C, bufsB, iaB, ibB)
        fire_gather(iaB, ibB, zaB, zbB, gsemB, hsemB)
        fire_tg(ebs(c0 + 1), tgB, tsemB)
        wait_gather(iaA, ibA, zaA, zbA, gsemA, hsemA)
        wait_tg(tgA, tsemA)
        acc = compute(_C, tgA, zaA, zbA, acc)

        @pl.when(j <= (_NCHUNK // 2 - 2))
        def _():
            fire_xy(ebs(c0 + 3), bufsB, ssemB)
            wait_xy(bufsA, ssemA)
            compute_idx(ebs(c0 + 2), _C, bufsA, iaA, ibA)
            fire_gather(iaA, ibA, zaA, zbA, gsemA, hsemA)
            fire_tg(ebs(c0 + 2), tgA, tsemA)
        wait_gather(iaB, ibB, zaB, zbB, gsemB, hsemB)
        wait_tg(tgB, tsemB)
        acc = compute(_C, tgB, zaB, zbB, acc)
        return acc

    acc = lax.fori_loop(0, _NCHUNK // 2, jbody, acc0)

    acc_v[...] = acc
    pltpu.sync_copy(acc_v, out.at[wid])


_depth_loss_sc = pl.kernel(
    _sc_body,
    out_type=jax.ShapeDtypeStruct((_NW, 16), jnp.float32),
    mesh=plsc.VectorSubcoreMesh(
        core_axis_name="c", subcore_axis_name="s", num_cores=2,
        num_subcores=16),
    scratch_types=(
        [pltpu.VMEM((_C,), jnp.int32)] * 8
        + [pltpu.VMEM((_C,), jnp.float32)] * 2
        + [pltpu.VMEM((_C,), jnp.int32)] * 4
        + [pltpu.VMEM((_C,), jnp.float32)] * 4
        + [pltpu.VMEM((_TG,), jnp.int32)] * 4
        + [pltpu.VMEM((_TG,), jnp.float32)]
        + [pltpu.VMEM((_TG,), jnp.int32)] * 2
        + [pltpu.VMEM((_TG,), jnp.float32)] * 2
        + [pltpu.VMEM((16,), jnp.float32)]
        + [pltpu.SemaphoreType.DMA] * 10
    ),
)


def kernel(output, x_A, y_A, x_B, y_B, ordinal_relation):
    img = output.reshape(_B * _H * _W)
    xa = x_A.reshape(_Q).astype(jnp.int32)
    ya = y_A.reshape(_Q).astype(jnp.int32)
    xb = x_B.reshape(_Q).astype(jnp.int32)
    yb = y_B.reshape(_Q).astype(jnp.int32)
    tg = ordinal_relation.reshape(_Q).astype(jnp.float32)
    partials = _depth_loss_sc(img, xa, ya, xb, yb, tg)
    return jnp.sum(partials) / _B


# speedup vs baseline: 1.0682x; 1.0125x over previous
import jax
import jax.numpy as jnp
from jax import lax
from jax.experimental import pallas as pl
from jax.experimental.pallas import tpu as pltpu
from jax.experimental.pallas import tpu_sc as plsc

_B, _H, _W, _P = 8, 512, 512, 100000
_Q = _B * _P
_NW = 32
_C = 2496
_NCHUNK = 10
_MAIN = _NW * _C * _NCHUNK
_TG = 128
_NTAIL = (_Q - _MAIN) // _TG


def _loss16(za, zb, t):
    d = za - zb
    m = t * t
    u = -(d * t)
    e = jnp.exp(-jnp.abs(u))
    s = e / (2.0 + e)
    s2 = s * s
    p = s2 * (1.0 / 7.0) + (1.0 / 5.0)
    p = s2 * p + (1.0 / 3.0)
    p = s2 * p + 1.0
    ln1pe = 2.0 * s * p
    sp = jnp.maximum(u, 0.0) + ln1pe
    return m * sp + (1.0 - m) * (d * d)


def _sc_body(img, xa, ya, xb, yb, tg, out,
             xaA, yaA, xbA, ybA, xaB, yaB, xbB, ybB, tgA, tgB,
             iaA, ibA, iaB, ibB, zaA, zbA, zaB, zbB,
             xt, yt, x2t, y2t, tgt_v, iat, ibt, zat, zbt, acc_v,
             ssemA, ssemB, tsemA, tsemB, gsemA, gsemB, tailsem,
             hsemA, hsemB, tailsem2):
    c_ax = lax.axis_index("c")
    s_ax = lax.axis_index("s")
    wid = c_ax * 16 + s_ax
    base = wid * (_C * _NCHUNK)

    xy_srcs = (xa, ya, xb, yb)

    def fire_xy(eb, bufs, sem):
        for src, dst in zip(xy_srcs, bufs):
            pltpu.async_copy(src.at[pl.ds(eb, _C)], dst, sem)

    def wait_xy(bufs, sem):
        for src, dst in zip(xy_srcs, bufs):
            pltpu.make_async_copy(src.at[pl.ds(0, _C)], dst, sem).wait()

    def fire_tg(eb, buf, sem):
        pltpu.async_copy(tg.at[pl.ds(eb, _C)], buf, sem)

    def wait_tg(buf, sem):
        pltpu.make_async_copy(tg.at[pl.ds(0, _C)], buf, sem).wait()

    def compute_idx(eb, n, bufs, ia_d, ib_d):
        xab, yab, xbb, ybb = bufs

        def ibody(i, _):
            sl = pl.ds(i * 16, 16)
            q = lax.iota(jnp.int32, 16) + (eb + i * 16)
            boff = lax.shift_left(lax.div(q, _P), 18)
            ia_d[sl] = boff + lax.shift_left(xab[sl], 9) + yab[sl]
            ib_d[sl] = boff + lax.shift_left(xbb[sl], 9) + ybb[sl]
            return 0
        lax.fori_loop(0, n // 16, ibody, 0, unroll=4)

    def fire_gather(ia_d, ib_d, za_d, zb_d, sem, hsem):
        pltpu.async_copy(img.at[ia_d], za_d, sem)
        pltpu.async_copy(img.at[ib_d], zb_d, hsem)

    def wait_gather(ia_d, ib_d, za_d, zb_d, sem, hsem):
        pltpu.make_async_copy(img.at[ia_d], za_d, sem).wait()
        pltpu.make_async_copy(img.at[ib_d], zb_d, hsem).wait()

    def compute(n, tg_d, za_d, zb_d, acc):
        def cbody(i, a):
            sl = pl.ds(i * 16, 16)
            return a + _loss16(za_d[sl], zb_d[sl], tg_d[sl])
        return lax.fori_loop(0, n // 16, cbody, acc, unroll=4)

    bufsA = (xaA, yaA, xbA, ybA)
    bufsB = (xaB, yaB, xbB, ybB)

    def ebs(c):
        return base + c * _C

    tw = jnp.clip(wid - (_NW - _NTAIL), 0, _NTAIL - 1)
    eb2 = _MAIN + tw * _TG
    pltpu.sync_copy(xa.at[pl.ds(eb2, _TG)], xt)
    pltpu.sync_copy(ya.at[pl.ds(eb2, _TG)], yt)
    pltpu.sync_copy(xb.at[pl.ds(eb2, _TG)], x2t)
    pltpu.sync_copy(yb.at[pl.ds(eb2, _TG)], y2t)
    pltpu.sync_copy(tg.at[pl.ds(eb2, _TG)], tgt_v)
    compute_idx(eb2, _TG, (xt, yt, x2t, y2t), iat, ibt)

    fire_xy(ebs(0), bufsA, ssemA)
    fire_xy(ebs(1), bufsB, ssemB)
    wait_xy(bufsA, ssemA)
    compute_idx(ebs(0), _C, bufsA, iaA, ibA)

    fire_gather(iat, ibt, zat, zbt, tailsem, tailsem2)
    fire_gather(iaA, ibA, zaA, zbA, gsemA, hsemA)
    fire_tg(ebs(0), tgA, tsemA)
    wait_gather(iat, ibt, zat, zbt, tailsem, tailsem2)
    tacc = compute(_TG, tgt_v, zat, zbt, jnp.zeros((16,), jnp.float32))
    wmask = (wid >= (_NW - _NTAIL)).astype(jnp.float32)
    acc0 = wmask * tacc

    def jbody(j, acc):
        c0 = 2 * j

        @pl.when(j <= (_NCHUNK // 2 - 2))
        def _():
            fire_xy(ebs(c0 + 2), bufsA, ssemA)
        wait_xy(bufsB, ssemB)
        compute_idx(ebs(c0 + 1), _C, bufsB, iaB, ibB)
        fire_gather(iaB, ibB, zaB, zbB, gsemB, hsemB)
        fire_tg(ebs(c0 + 1), tgB, tsemB)
        wait_gather(iaA, ibA, zaA, zbA, gsemA, hsemA)
        wait_tg(tgA, tsemA)
        acc = compute(_C, tgA, zaA, zbA, acc)

        @pl.when(j <= (_NCHUNK // 2 - 2))
        def _():
            fire_xy(ebs(c0 + 3), bufsB, ssemB)
            wait_xy(bufsA, ssemA)
            compute_idx(ebs(c0 + 2), _C, bufsA, iaA, ibA)
            fire_gather(iaA, ibA, zaA, zbA, gsemA, hsemA)
            fire_tg(ebs(c0 + 2), tgA, tsemA)
        wait_gather(iaB, ibB, zaB, zbB, gsemB, hsemB)
        wait_tg(tgB, tsemB)
        acc = compute(_C, tgB, zaB, zbB, acc)
        return acc

    acc = lax.fori_loop(0, _NCHUNK // 2, jbody, acc0)

    acc_v[...] = acc
    pltpu.sync_copy(acc_v, out.at[wid])


_depth_loss_sc = pl.kernel(
    _sc_body,
    out_type=jax.ShapeDtypeStruct((_NW, 16), jnp.float32),
    mesh=plsc.VectorSubcoreMesh(
        core_axis_name="c", subcore_axis_name="s", num_cores=2,
        num_subcores=16),
    scratch_types=(
        [pltpu.VMEM((_C,), jnp.int32)] * 8
        + [pltpu.VMEM((_C,), jnp.float32)] * 2
        + [pltpu.VMEM((_C,), jnp.int32)] * 4
        + [pltpu.VMEM((_C,), jnp.float32)] * 4
        + [pltpu.VMEM((_TG,), jnp.int32)] * 4
        + [pltpu.VMEM((_TG,), jnp.float32)]
        + [pltpu.VMEM((_TG,), jnp.int32)] * 2
        + [pltpu.VMEM((_TG,), jnp.float32)] * 2
        + [pltpu.VMEM((16,), jnp.float32)]
        + [pltpu.SemaphoreType.DMA] * 10
    ),
)


def kernel(output, x_A, y_A, x_B, y_B, ordinal_relation):
    img = output.reshape(_B * _H * _W)
    xa = x_A.reshape(_Q).astype(jnp.int32)
    ya = y_A.reshape(_Q).astype(jnp.int32)
    xb = x_B.reshape(_Q).astype(jnp.int32)
    yb = y_B.reshape(_Q).astype(jnp.int32)
    tg = ordinal_relation.reshape(_Q).astype(jnp.float32)
    partials = _depth_loss_sc(img, xa, ya, xb, yb, tg)
    return jnp.sum(partials) / _B
